# baseline (speedup 1.0000x reference)
import functools

import jax
import jax.numpy as jnp
from jax import lax
from jax.experimental import pallas as pl
from jax.experimental.pallas import tpu as pltpu
from jax.experimental.pallas import tpu_sc as plsc

SEQ = 4
NBATCH = 4
NN = 10000
N = NBATCH * NN
E = 640000
FIN = 128
NHID = 2
RNN_FEAT = NN // 3
RNN_HIDDEN = 3
EPS = 1e-5

NC, NS = 2, 16
NW = NC * NS
RPT = 2560
NPAD = NS * RPT
EPAD = 655360
EPW = EPAD // NW
CH = 1024
OUTER = EPW // CH


def _sc_prop(F):
    mesh = plsc.VectorSubcoreMesh(core_axis_name="c", subcore_axis_name="s")

    def body(zflat, srcf, dstf, out_hbm, srcb, dstb, ztab, acc):
        c = lax.axis_index("c")
        s = lax.axis_index("s")
        wid = s * NC + c
        pltpu.sync_copy(srcf.at[pl.ds(wid * EPW, EPW)], srcb)
        pltpu.sync_copy(dstf.at[pl.ds(wid * EPW, EPW)], dstb)
        for f in range(F):
            pltpu.sync_copy(zflat.at[pl.ds(f * NPAD, NPAD)], ztab)

            @plsc.parallel_loop(0, NPAD, 16, unroll=8)
            def _(i):
                acc[pl.ds(i, 16)] = jnp.zeros((16,), jnp.float32)

            @plsc.parallel_loop(0, EPW, 16, unroll=8)
            def _(i):
                s16 = srcb[pl.ds(i, 16)]
                d16 = dstb[pl.ds(i, 16)]
                v = plsc.load_gather(ztab, [s16])
                plsc.addupdate_scatter(acc, [d16], v)

            pltpu.sync_copy(acc, out_hbm.at[pl.ds((f * NW + wid) * NPAD, NPAD)])

    return pl.kernel(
        body,
        out_type=jax.ShapeDtypeStruct((F * NW * NPAD,), jnp.float32),
        mesh=mesh,
        compiler_params=pltpu.CompilerParams(needs_layout_passes=False),
        scratch_types=[
            pltpu.VMEM((EPW,), jnp.int32),
            pltpu.VMEM((EPW,), jnp.int32),
            pltpu.VMEM((NPAD,), jnp.float32),
            pltpu.VMEM((NPAD,), jnp.float32),
        ],
    )


def _sc_deg():
    mesh = plsc.VectorSubcoreMesh(core_axis_name="c", subcore_axis_name="s")

    def body(dstf, out_hbm, dstb, acc):
        c = lax.axis_index("c")
        s = lax.axis_index("s")
        wid = s * NC + c
        pltpu.sync_copy(dstf.at[pl.ds(wid * EPW, EPW)], dstb)

        @plsc.parallel_loop(0, NPAD, 16, unroll=8)
        def _(i):
            acc[pl.ds(i, 16)] = jnp.zeros((16,), jnp.float32)

        ones16 = jnp.ones((16,), jnp.float32)

        @plsc.parallel_loop(0, EPW, 16, unroll=8)
        def _(i):
            d16 = dstb[pl.ds(i, 16)]
            plsc.addupdate_scatter(acc, [d16], ones16)

        pltpu.sync_copy(acc, out_hbm.at[pl.ds(wid * NPAD, NPAD)])

    return pl.kernel(
        body,
        out_type=jax.ShapeDtypeStruct((NW * NPAD,), jnp.float32),
        mesh=mesh,
        compiler_params=pltpu.CompilerParams(needs_layout_passes=False),
        scratch_types=[
            pltpu.VMEM((EPW,), jnp.int32),
            pltpu.VMEM((NPAD,), jnp.float32),
        ],
    )


_BN = 4096
_NB = NPAD // _BN


def _tc_xw(x, W1):
    def body(xref, wref, oref):
        parts = [
            lax.dot_general(wref[t], xref[t], (((0,), (1,)), ((), ())),
                            preferred_element_type=jnp.float32)
            for t in range(SEQ)
        ]
        oref[...] = jnp.concatenate(parts, axis=0)

    return pl.pallas_call(
        body,
        grid=(_NB,),
        in_specs=[
            pl.BlockSpec((SEQ, _BN, FIN), lambda i: (0, i, 0)),
            pl.BlockSpec((SEQ, FIN, NHID), lambda i: (0, 0, 0)),
        ],
        out_specs=pl.BlockSpec((SEQ * NHID, _BN), lambda i: (0, i)),
        out_shape=jax.ShapeDtypeStruct((SEQ * NHID, NPAD), jnp.float32),
    )(x, W1)


def _tc_dinv_z1(degr, xw):
    def body(dref, xwref, dinv_o, z1_o):
        deg = jnp.sum(dref[...], axis=0, keepdims=True) + 1.0
        dinv = lax.rsqrt(deg)
        dinv_o[...] = dinv
        z1_o[...] = xwref[...] * dinv

    return pl.pallas_call(
        body,
        grid=(_NB,),
        in_specs=[
            pl.BlockSpec((NW, _BN), lambda i: (0, i)),
            pl.BlockSpec((SEQ * NHID, _BN), lambda i: (0, i)),
        ],
        out_specs=[
            pl.BlockSpec((1, _BN), lambda i: (0, i)),
            pl.BlockSpec((SEQ * NHID, _BN), lambda i: (0, i)),
        ],
        out_shape=[
            jax.ShapeDtypeStruct((1, NPAD), jnp.float32),
            jax.ShapeDtypeStruct((SEQ * NHID, NPAD), jnp.float32),
        ],
    )(degr, xw)


def _tc_z2(p1flat, z1, dinv, b1col, W2):
    F = SEQ * NHID

    def body(*refs):
        prefs = refs[:F]
        z1ref, dref, b1ref, w2ref, z2_o, accs = refs[F:]
        w = pl.program_id(1)

        @pl.when(w == 0)
        def _():
            accs[...] = jnp.zeros((F, _BN), jnp.float32)

        accs[...] += jnp.concatenate([p[...][None] for p in prefs], axis=0)

        @pl.when(w == NW - 1)
        def _():
            acc = accs[...] + z1ref[...]
            dinv = dref[...]
            out1 = jax.nn.sigmoid(acc * dinv + b1ref[...])
            parts = []
            for t in range(SEQ):
                o2 = out1[NHID * t:NHID * (t + 1)]
                parts.append(jnp.sum(o2 * w2ref[t], axis=0, keepdims=True))
            z2_o[...] = jnp.concatenate(parts, axis=0) * dinv

    return pl.pallas_call(
        body,
        grid=(_NB, NW),
        in_specs=[
            pl.BlockSpec((_BN,), lambda nb, w, f=f: ((f * NW + w) * _NB + nb,))
            for f in range(F)
        ] + [
            pl.BlockSpec((F, _BN), lambda nb, w: (0, nb)),
            pl.BlockSpec((1, _BN), lambda nb, w: (0, nb)),
            pl.BlockSpec((F, 1), lambda nb, w: (0, 0)),
            pl.BlockSpec((SEQ, NHID, 1), lambda nb, w: (0, 0, 0)),
        ],
        out_specs=pl.BlockSpec((SEQ, _BN), lambda nb, w: (0, nb)),
        out_shape=jax.ShapeDtypeStruct((SEQ, NPAD), jnp.float32),
        scratch_shapes=[pltpu.VMEM((F, _BN), jnp.float32)],
    )(*([p1flat] * F), z1, dinv, b1col, W2)


def _tc_h2(p2flat, z2, dinv, b2col):
    F = SEQ

    def body(*refs):
        prefs = refs[:F]
        z2ref, dref, b2ref, h2_o, accs = refs[F:]
        w = pl.program_id(1)

        @pl.when(w == 0)
        def _():
            accs[...] = jnp.zeros((F, _BN), jnp.float32)

        accs[...] += jnp.concatenate([p[...][None] for p in prefs], axis=0)

        @pl.when(w == NW - 1)
        def _():
            acc = accs[...] + z2ref[...]
            h2_o[...] = jax.nn.sigmoid(acc * dref[...] + b2ref[...])

    return pl.pallas_call(
        body,
        grid=(_NB, NW),
        in_specs=[
            pl.BlockSpec((_BN,), lambda nb, w, f=f: ((f * NW + w) * _NB + nb,))
            for f in range(F)
        ] + [
            pl.BlockSpec((F, _BN), lambda nb, w: (0, nb)),
            pl.BlockSpec((1, _BN), lambda nb, w: (0, nb)),
            pl.BlockSpec((F, 1), lambda nb, w: (0, 0)),
        ],
        out_specs=pl.BlockSpec((SEQ, _BN), lambda nb, w: (0, nb)),
        out_shape=jax.ShapeDtypeStruct((SEQ, NPAD), jnp.float32),
        scratch_shapes=[pltpu.VMEM((F, _BN), jnp.float32)],
    )(*([p2flat] * F), z2, dinv, b2col)


_BJ = 128


def _tc_lin_bn(h2t, Wlt, scale, shift):
    jb = (RNN_FEAT + _BJ - 1) // _BJ

    def body(href, wref, sref, shref, y_o):
        parts = []
        for t in range(SEQ):
            yt = lax.dot_general(
                href[t], wref[:, t, :], (((1,), (1,)), ((), ())),
                preferred_element_type=jnp.float32,
            )
            parts.append(yt[None])
        y = jnp.concatenate(parts, axis=0)
        y_o[...] = y * sref[...] + shref[...]

    return pl.pallas_call(
        body,
        grid=(jb,),
        in_specs=[
            pl.BlockSpec((SEQ, NBATCH, NN), lambda j: (0, 0, 0)),
            pl.BlockSpec((_BJ, SEQ, NN), lambda j: (j, 0, 0)),
            pl.BlockSpec((SEQ, 1, _BJ), lambda j: (0, 0, j)),
            pl.BlockSpec((SEQ, 1, _BJ), lambda j: (0, 0, j)),
        ],
        out_specs=pl.BlockSpec((SEQ, NBATCH, _BJ), lambda j: (0, 0, j)),
        out_shape=jax.ShapeDtypeStruct((SEQ, NBATCH, RNN_FEAT), jnp.float32),
    )(h2t, Wlt, scale, shift)


def _tc_gru(y, Wih0, Whh0, bih0, bhh0, Wih1, Whh1, bih1, bhh1, dW, db):
    ct = (((1,), (1,)), ((), ()))

    def gru_steps(gi, Whh, bhh):
        h = jnp.zeros((NBATCH, RNN_HIDDEN), jnp.float32)
        ys = []
        for t in range(SEQ):
            gh = lax.dot_general(h, Whh, ct, preferred_element_type=jnp.float32) + bhh
            git = gi[NBATCH * t:NBATCH * (t + 1)]
            r = jax.nn.sigmoid(git[:, 0:3] + gh[:, 0:3])
            z = jax.nn.sigmoid(git[:, 3:6] + gh[:, 3:6])
            n = jnp.tanh(git[:, 6:9] + r * gh[:, 6:9])
            h = (1.0 - z) * n + z * h
            ys.append(h)
        return ys, h

    def body(yref, wi0, wh0, bi0, bh0, wi1, wh1, bi1, bh1, dwref, dbref,
             out_o, hn_o):
        yf = yref[...].reshape(SEQ * NBATCH, RNN_FEAT)
        gi0 = lax.dot_general(yf, wi0[...], ct, preferred_element_type=jnp.float32) + bi0[...]
        ys0, hT0 = gru_steps(gi0, wh0[...], bh0[...])
        y0f = jnp.concatenate(ys0, axis=0)
        gi1 = lax.dot_general(y0f, wi1[...], ct, preferred_element_type=jnp.float32) + bi1[...]
        ys1, hT1 = gru_steps(gi1, wh1[...], bh1[...])
        y1f = jnp.concatenate(ys1, axis=0)
        out = lax.dot_general(y1f, dwref[...], ct, preferred_element_type=jnp.float32) + dbref[...]
        out_o[...] = out.reshape(SEQ, NBATCH, 128)
        hn_o[...] = jnp.concatenate([hT0[None], hT1[None]], axis=0)

    return pl.pallas_call(
        body,
        out_shape=[
            jax.ShapeDtypeStruct((SEQ, NBATCH, 128), jnp.float32),
            jax.ShapeDtypeStruct((2, NBATCH, RNN_HIDDEN), jnp.float32),
        ],
    )(y, Wih0, Whh0, bih0, bhh0, Wih1, Whh1, bih1, bhh1, dW, db)


def kernel(x, edge_index, W1, b1, W2, b2, Wlin, blin, bn_gamma, bn_beta,
           bn_mean, bn_var, gWih0, gWhh0, gbih0, gbhh0, gWih1, gWhh1,
           gbih1, gbhh1, dec_W, dec_b):
    src = edge_index[0]
    dst = edge_index[1]
    npad_e = EPAD - E
    padidx = N + (lax.iota(jnp.int32, npad_e) % (NPAD - N))
    srcf = jnp.concatenate([src, padidx])
    dstf = jnp.concatenate([dst, padidx])

    degr = _sc_deg()(dstf).reshape(NW, NPAD)
    xw = _tc_xw(x, W1)
    dinv, z1 = _tc_dinv_z1(degr, xw)
    p1 = _sc_prop(SEQ * NHID)(z1.reshape(-1), srcf, dstf)
    z2 = _tc_z2(p1, z1, dinv, b1.reshape(SEQ * NHID, 1), W2)
    p2 = _sc_prop(SEQ)(z2.reshape(-1), srcf, dstf)
    h2 = _tc_h2(p2, z2, dinv, b2.reshape(SEQ, 1))
    h2t = h2[:, :N].reshape(SEQ, NBATCH, NN)
    scale = (bn_gamma * lax.rsqrt(bn_var + EPS))[:, None, :]
    shift = ((blin - bn_mean) * scale[:, 0] + bn_beta)[:, None, :]
    y = _tc_lin_bn(h2t, jnp.transpose(Wlin, (1, 0, 2)), scale, shift)
    dec_Wp = jnp.concatenate(
        [dec_W, jnp.zeros((127, RNN_HIDDEN), jnp.float32)], axis=0)
    dec_bp = jnp.concatenate([dec_b, jnp.zeros((127,), jnp.float32)])
    outp, hn = _tc_gru(y, gWih0, gWhh0, gbih0, gbhh0, gWih1, gWhh1,
                       gbih1, gbhh1, dec_Wp, dec_bp)
    return (outp[:, :, :1], hn)


# speedup vs baseline: 1.4659x; 1.4659x over previous
import functools

import jax
import jax.numpy as jnp
from jax import lax
from jax.experimental import pallas as pl
from jax.experimental.pallas import tpu as pltpu
from jax.experimental.pallas import tpu_sc as plsc

SEQ = 4
NBATCH = 4
NN = 10000
N = NBATCH * NN
E = 640000
FIN = 128
NHID = 2
RNN_FEAT = NN // 3
RNN_HIDDEN = 3
EPS = 1e-5

NC, NS = 2, 16
NW = NC * NS
RPT = 2560
NPAD = NS * RPT
EPAD = 655360
EPW = EPAD // NW
CH = 1024
OUTER = EPW // CH


def _sc_prop(F):
    mesh = plsc.VectorSubcoreMesh(core_axis_name="c", subcore_axis_name="s")

    def body(zflat, srcf, dstf, out_hbm, srcb, dstb, ztab, acc):
        c = lax.axis_index("c")
        s = lax.axis_index("s")
        wid = s * NC + c
        pltpu.sync_copy(srcf.at[pl.ds(wid * EPW, EPW)], srcb)
        pltpu.sync_copy(dstf.at[pl.ds(wid * EPW, EPW)], dstb)
        for f in range(F):
            pltpu.sync_copy(zflat.at[pl.ds(f * NPAD, NPAD)], ztab)

            @plsc.parallel_loop(0, NPAD, 16, unroll=8)
            def _(i):
                acc[pl.ds(i, 16)] = jnp.zeros((16,), jnp.float32)

            @plsc.parallel_loop(0, EPW, 16, unroll=8)
            def _(i):
                s16 = srcb[pl.ds(i, 16)]
                d16 = dstb[pl.ds(i, 16)]
                v = plsc.load_gather(ztab, [s16])
                plsc.addupdate_scatter(acc, [d16], v)

            pltpu.sync_copy(acc, out_hbm.at[pl.ds((f * NW + wid) * NPAD, NPAD)])

    return pl.kernel(
        body,
        out_type=jax.ShapeDtypeStruct((F * NW * NPAD,), jnp.float32),
        mesh=mesh,
        compiler_params=pltpu.CompilerParams(needs_layout_passes=False),
        scratch_types=[
            pltpu.VMEM((EPW,), jnp.int32),
            pltpu.VMEM((EPW,), jnp.int32),
            pltpu.VMEM((NPAD,), jnp.float32),
            pltpu.VMEM((NPAD,), jnp.float32),
        ],
    )


def _sc_deg():
    mesh = plsc.VectorSubcoreMesh(core_axis_name="c", subcore_axis_name="s")

    def body(dstf, out_hbm, dstb, acc):
        c = lax.axis_index("c")
        s = lax.axis_index("s")
        wid = s * NC + c
        pltpu.sync_copy(dstf.at[pl.ds(wid * EPW, EPW)], dstb)

        @plsc.parallel_loop(0, NPAD, 16, unroll=8)
        def _(i):
            acc[pl.ds(i, 16)] = jnp.zeros((16,), jnp.float32)

        ones16 = jnp.ones((16,), jnp.float32)

        @plsc.parallel_loop(0, EPW, 16, unroll=8)
        def _(i):
            d16 = dstb[pl.ds(i, 16)]
            plsc.addupdate_scatter(acc, [d16], ones16)

        pltpu.sync_copy(acc, out_hbm.at[pl.ds(wid * NPAD, NPAD)])

    return pl.kernel(
        body,
        out_type=jax.ShapeDtypeStruct((NW * NPAD,), jnp.float32),
        mesh=mesh,
        compiler_params=pltpu.CompilerParams(needs_layout_passes=False),
        scratch_types=[
            pltpu.VMEM((EPW,), jnp.int32),
            pltpu.VMEM((NPAD,), jnp.float32),
        ],
    )


_BN = 4096
_NB = NPAD // _BN


def _tc_xw(x, W1):
    def body(xref, wref, oref):
        parts = [
            lax.dot_general(wref[t], xref[t], (((0,), (1,)), ((), ())),
                            preferred_element_type=jnp.float32)
            for t in range(SEQ)
        ]
        oref[...] = jnp.concatenate(parts, axis=0)

    return pl.pallas_call(
        body,
        grid=(_NB,),
        in_specs=[
            pl.BlockSpec((SEQ, _BN, FIN), lambda i: (0, i, 0)),
            pl.BlockSpec((SEQ, FIN, NHID), lambda i: (0, 0, 0)),
        ],
        out_specs=pl.BlockSpec((SEQ * NHID, _BN), lambda i: (0, i)),
        out_shape=jax.ShapeDtypeStruct((SEQ * NHID, NPAD), jnp.float32),
    )(x, W1)


def _tc_dinv_z1(degr, xw):
    def body(dref, xwref, dinv_o, z1_o):
        deg = jnp.sum(dref[...], axis=0, keepdims=True) + 1.0
        dinv = lax.rsqrt(deg)
        dinv_o[...] = dinv
        z1_o[...] = xwref[...] * dinv

    return pl.pallas_call(
        body,
        grid=(_NB,),
        in_specs=[
            pl.BlockSpec((NW, _BN), lambda i: (0, i)),
            pl.BlockSpec((SEQ * NHID, _BN), lambda i: (0, i)),
        ],
        out_specs=[
            pl.BlockSpec((1, _BN), lambda i: (0, i)),
            pl.BlockSpec((SEQ * NHID, _BN), lambda i: (0, i)),
        ],
        out_shape=[
            jax.ShapeDtypeStruct((1, NPAD), jnp.float32),
            jax.ShapeDtypeStruct((SEQ * NHID, NPAD), jnp.float32),
        ],
    )(degr, xw)


def _tc_z2(p1, z1, dinv, b1col, W2):
    def body(pref, z1ref, dref, b1ref, w2ref, z2_o):
        acc = jnp.sum(pref[...], axis=1) + z1ref[...]
        dinv = dref[...]
        out1 = jax.nn.sigmoid(acc * dinv + b1ref[...])
        parts = []
        for t in range(SEQ):
            o2 = out1[NHID * t:NHID * (t + 1)]
            parts.append(jnp.sum(o2 * w2ref[t], axis=0, keepdims=True))
        z2_o[...] = jnp.concatenate(parts, axis=0) * dinv

    return pl.pallas_call(
        body,
        grid=(_NB,),
        in_specs=[
            pl.BlockSpec((SEQ * NHID, NW, _BN), lambda i: (0, 0, i)),
            pl.BlockSpec((SEQ * NHID, _BN), lambda i: (0, i)),
            pl.BlockSpec((1, _BN), lambda i: (0, i)),
            pl.BlockSpec((SEQ * NHID, 1), lambda i: (0, 0)),
            pl.BlockSpec((SEQ, NHID, 1), lambda i: (0, 0, 0)),
        ],
        out_specs=pl.BlockSpec((SEQ, _BN), lambda i: (0, i)),
        out_shape=jax.ShapeDtypeStruct((SEQ, NPAD), jnp.float32),
    )(p1, z1, dinv, b1col, W2)


def _tc_h2(p2, z2, dinv, b2col):
    def body(pref, z2ref, dref, b2ref, h2_o):
        acc = jnp.sum(pref[...], axis=1) + z2ref[...]
        h2_o[...] = jax.nn.sigmoid(acc * dref[...] + b2ref[...])

    return pl.pallas_call(
        body,
        grid=(_NB,),
        in_specs=[
            pl.BlockSpec((SEQ, NW, _BN), lambda i: (0, 0, i)),
            pl.BlockSpec((SEQ, _BN), lambda i: (0, i)),
            pl.BlockSpec((1, _BN), lambda i: (0, i)),
            pl.BlockSpec((SEQ, 1), lambda i: (0, 0)),
        ],
        out_specs=pl.BlockSpec((SEQ, _BN), lambda i: (0, i)),
        out_shape=jax.ShapeDtypeStruct((SEQ, NPAD), jnp.float32),
    )(p2, z2, dinv, b2col)


_BJ = 128


def _tc_lin_bn(h2t, Wlt, scale, shift):
    jb = (RNN_FEAT + _BJ - 1) // _BJ

    def body(href, wref, sref, shref, y_o):
        parts = []
        for t in range(SEQ):
            yt = lax.dot_general(
                href[t], wref[:, t, :], (((1,), (1,)), ((), ())),
                preferred_element_type=jnp.float32,
            )
            parts.append(yt[None])
        y = jnp.concatenate(parts, axis=0)
        y_o[...] = y * sref[...] + shref[...]

    return pl.pallas_call(
        body,
        grid=(jb,),
        in_specs=[
            pl.BlockSpec((SEQ, NBATCH, NN), lambda j: (0, 0, 0)),
            pl.BlockSpec((_BJ, SEQ, NN), lambda j: (j, 0, 0)),
            pl.BlockSpec((SEQ, 1, _BJ), lambda j: (0, 0, j)),
            pl.BlockSpec((SEQ, 1, _BJ), lambda j: (0, 0, j)),
        ],
        out_specs=pl.BlockSpec((SEQ, NBATCH, _BJ), lambda j: (0, 0, j)),
        out_shape=jax.ShapeDtypeStruct((SEQ, NBATCH, RNN_FEAT), jnp.float32),
    )(h2t, Wlt, scale, shift)


def _tc_gru(y, Wih0, Whh0, bih0, bhh0, Wih1, Whh1, bih1, bhh1, dW, db):
    ct = (((1,), (1,)), ((), ()))

    def gru_steps(gi, Whh, bhh):
        h = jnp.zeros((NBATCH, RNN_HIDDEN), jnp.float32)
        ys = []
        for t in range(SEQ):
            gh = lax.dot_general(h, Whh, ct, preferred_element_type=jnp.float32) + bhh
            git = gi[NBATCH * t:NBATCH * (t + 1)]
            r = jax.nn.sigmoid(git[:, 0:3] + gh[:, 0:3])
            z = jax.nn.sigmoid(git[:, 3:6] + gh[:, 3:6])
            n = jnp.tanh(git[:, 6:9] + r * gh[:, 6:9])
            h = (1.0 - z) * n + z * h
            ys.append(h)
        return ys, h

    def body(yref, wi0, wh0, bi0, bh0, wi1, wh1, bi1, bh1, dwref, dbref,
             out_o, hn_o):
        yf = yref[...].reshape(SEQ * NBATCH, RNN_FEAT)
        gi0 = lax.dot_general(yf, wi0[...], ct, preferred_element_type=jnp.float32) + bi0[...]
        ys0, hT0 = gru_steps(gi0, wh0[...], bh0[...])
        y0f = jnp.concatenate(ys0, axis=0)
        gi1 = lax.dot_general(y0f, wi1[...], ct, preferred_element_type=jnp.float32) + bi1[...]
        ys1, hT1 = gru_steps(gi1, wh1[...], bh1[...])
        y1f = jnp.concatenate(ys1, axis=0)
        out = lax.dot_general(y1f, dwref[...], ct, preferred_element_type=jnp.float32) + dbref[...]
        out_o[...] = out.reshape(SEQ, NBATCH, 128)
        hn_o[...] = jnp.concatenate([hT0[None], hT1[None]], axis=0)

    return pl.pallas_call(
        body,
        out_shape=[
            jax.ShapeDtypeStruct((SEQ, NBATCH, 128), jnp.float32),
            jax.ShapeDtypeStruct((2, NBATCH, RNN_HIDDEN), jnp.float32),
        ],
    )(y, Wih0, Whh0, bih0, bhh0, Wih1, Whh1, bih1, bhh1, dW, db)


def kernel(x, edge_index, W1, b1, W2, b2, Wlin, blin, bn_gamma, bn_beta,
           bn_mean, bn_var, gWih0, gWhh0, gbih0, gbhh0, gWih1, gWhh1,
           gbih1, gbhh1, dec_W, dec_b):
    src = edge_index[0]
    dst = edge_index[1]
    npad_e = EPAD - E
    padidx = N + (lax.iota(jnp.int32, npad_e) % (NPAD - N))
    srcf = jnp.concatenate([src, padidx])
    dstf = jnp.concatenate([dst, padidx])

    degr = _sc_deg()(dstf).reshape(NW, NPAD)
    xw = _tc_xw(x, W1)
    dinv, z1 = _tc_dinv_z1(degr, xw)
    p1 = _sc_prop(SEQ * NHID)(z1.reshape(-1), srcf, dstf)
    p1r = p1.reshape(SEQ * NHID, NW, NPAD)
    z2 = _tc_z2(p1r, z1, dinv, b1.reshape(SEQ * NHID, 1), W2)
    p2 = _sc_prop(SEQ)(z2.reshape(-1), srcf, dstf)
    p2r = p2.reshape(SEQ, NW, NPAD)
    h2 = _tc_h2(p2r, z2, dinv, b2.reshape(SEQ, 1))
    h2t = h2[:, :N].reshape(SEQ, NBATCH, NN)
    scale = (bn_gamma * lax.rsqrt(bn_var + EPS))[:, None, :]
    shift = ((blin - bn_mean) * scale[:, 0] + bn_beta)[:, None, :]
    y = _tc_lin_bn(h2t, jnp.transpose(Wlin, (1, 0, 2)), scale, shift)
    dec_Wp = jnp.concatenate(
        [dec_W, jnp.zeros((127, RNN_HIDDEN), jnp.float32)], axis=0)
    dec_bp = jnp.concatenate([dec_b, jnp.zeros((127,), jnp.float32)])
    outp, hn = _tc_gru(y, gWih0, gWhh0, gbih0, gbhh0, gWih1, gWhh1,
                       gbih1, gbhh1, dec_Wp, dec_bp)
    return (outp[:, :, :1], hn)
